# baseline (speedup 1.0000x reference)
import jax
import jax.numpy as jnp
from jax.experimental import pallas as pl

_NUM_EMB = 1024
_EMB_DIM = 64
_COMMIT = 0.25
_BB = 4


def _vq_kernel(z_ref, cb_ref, iota_col_ref, zq_ref, idx_ref, loss_ref):
    cb = cb_ref[...]
    iota_col = iota_col_ref[...]
    c_sq = jnp.sum(cb * cb, axis=1)[None, :]
    cb2 = cb + cb

    partial = jnp.zeros((1, 1), jnp.float32)
    for j in range(_BB):
        z_t = z_ref[j]
        flat = z_t.T

        f_sq = jnp.sum(flat * flat, axis=1, keepdims=True)
        dot2 = jax.lax.dot_general(
            flat, cb2, (((1,), (1,)), ((), ())),
            preferred_element_type=jnp.float32)
        sq = (f_sq + c_sq) - dot2
        dists = jnp.sqrt(jnp.maximum(sq, 0.0))

        minval = jnp.min(dists, axis=1, keepdims=True)
        idx = jnp.min(
            jnp.where(dists == minval, iota_col.T, _NUM_EMB), axis=1)
        idx_ref[j] = idx.reshape(8, 128)

        onehot_t = jnp.where(iota_col == idx[None, :], 1.0, 0.0)
        zq_t = jax.lax.dot_general(
            cb, onehot_t, (((0,), (0,)), ((), ())),
            preferred_element_type=jnp.float32,
            precision=jax.lax.Precision.HIGHEST)
        zq_ref[j] = zq_t

        partial += jnp.sum(minval * minval).reshape(1, 1)

    @pl.when(pl.program_id(0) == 0)
    def _init():
        loss_ref[...] = jnp.zeros((1, 1), jnp.float32)

    loss_ref[...] += partial


def kernel(z_e, codebook):
    b, c, h, w = z_e.shape
    n_pos = b * h * w
    hw = h * w
    z3 = z_e.reshape(b, c, hw)
    iota_col = jax.lax.broadcasted_iota(jnp.int32, (_NUM_EMB, 1), 0)

    zq3, idx3, loss_sum = pl.pallas_call(
        _vq_kernel,
        grid=(b // _BB,),
        in_specs=[
            pl.BlockSpec((_BB, c, hw), lambda i: (i, 0, 0)),
            pl.BlockSpec((_NUM_EMB, _EMB_DIM), lambda i: (0, 0)),
            pl.BlockSpec((_NUM_EMB, 1), lambda i: (0, 0)),
        ],
        out_specs=[
            pl.BlockSpec((_BB, c, hw), lambda i: (i, 0, 0)),
            pl.BlockSpec((_BB, 8, 128), lambda i: (i, 0, 0)),
            pl.BlockSpec((1, 1), lambda i: (0, 0)),
        ],
        out_shape=[
            jax.ShapeDtypeStruct((b, c, hw), jnp.float32),
            jax.ShapeDtypeStruct((b, 8, 128), jnp.int32),
            jax.ShapeDtypeStruct((1, 1), jnp.float32),
        ],
    )(z3, codebook, iota_col)

    loss = loss_sum[0, 0] * ((1.0 + _COMMIT) / (n_pos * _EMB_DIM))
    indices = idx3.reshape(n_pos)
    zq = zq3.reshape(b, c, h, w)
    return (zq, loss, indices)


# speedup vs baseline: 1.1996x; 1.1996x over previous
import functools

import jax
import jax.numpy as jnp
from jax import lax
from jax.experimental import pallas as pl
from jax.experimental.pallas import tpu as pltpu
from jax.experimental.pallas import tpu_sc as plsc

_NUM_EMB = 1024
_EMB_DIM = 64
_COMMIT = 0.25
_BB = 4
_HW = 1024
_L = 16


def _vq_tc_kernel(z_ref, cb_ref, iota_col_ref, idx_ref, loss_ref):
    cb = cb_ref[...]
    iota_col = iota_col_ref[...]
    c_sq = jnp.sum(cb * cb, axis=1)[None, :]
    cb2 = cb + cb

    partial = jnp.zeros((1, 1), jnp.float32)
    for j in range(_BB):
        z_t = z_ref[j]
        flat = z_t.T

        f_sq = jnp.sum(flat * flat, axis=1, keepdims=True)
        dot2 = jax.lax.dot_general(
            flat, cb2, (((1,), (1,)), ((), ())),
            preferred_element_type=jnp.float32)
        sq = (f_sq + c_sq) - dot2
        dists = jnp.sqrt(jnp.maximum(sq, 0.0))

        minval = jnp.min(dists, axis=1, keepdims=True)
        idx = jnp.min(
            jnp.where(dists == minval, iota_col.T, _NUM_EMB), axis=1)
        idx_ref[j] = idx.reshape(8, 128)

        partial += jnp.sum(minval * minval).reshape(1, 1)

    @pl.when(pl.program_id(0) == 0)
    def _init():
        loss_ref[...] = jnp.zeros((1, 1), jnp.float32)

    loss_ref[...] += partial


def _indices_and_loss(z3, codebook, b, c, hw):
    iota_col = jax.lax.broadcasted_iota(jnp.int32, (_NUM_EMB, 1), 0)
    idx3, loss_sum = pl.pallas_call(
        _vq_tc_kernel,
        grid=(b // _BB,),
        in_specs=[
            pl.BlockSpec((_BB, c, hw), lambda i: (i, 0, 0)),
            pl.BlockSpec((_NUM_EMB, _EMB_DIM), lambda i: (0, 0)),
            pl.BlockSpec((_NUM_EMB, 1), lambda i: (0, 0)),
        ],
        out_specs=[
            pl.BlockSpec((_BB, 8, 128), lambda i: (i, 0, 0)),
            pl.BlockSpec((1, 1), lambda i: (0, 0)),
        ],
        out_shape=[
            jax.ShapeDtypeStruct((b, 8, 128), jnp.int32),
            jax.ShapeDtypeStruct((1, 1), jnp.float32),
        ],
    )(z3, codebook, iota_col)
    return idx3, loss_sum


def _make_sc_gather(b):
    mesh = plsc.VectorSubcoreMesh(core_axis_name="c", subcore_axis_name="s")

    @functools.partial(
        pl.kernel, mesh=mesh,
        out_type=jax.ShapeDtypeStruct((b, _EMB_DIM, _HW), jnp.float32),
        scratch_types=[
            pltpu.VMEM((_EMB_DIM * _NUM_EMB,), jnp.float32),
            pltpu.VMEM((_HW,), jnp.int32),
            pltpu.VMEM((_EMB_DIM, _HW // 2), jnp.float32),
        ],
        compiler_params=pltpu.CompilerParams(needs_layout_passes=False),
    )
    def sc_gather(cbt_hbm, idx_hbm, out_hbm, cbt_v, idx_v, out_v):
        wid = lax.axis_index("s") * 2 + lax.axis_index("c")
        pltpu.sync_copy(cbt_hbm, cbt_v)
        pltpu.sync_copy(idx_hbm.at[wid], idx_v)

        def half(h):
            def chunk(i, carry):
                vidx = idx_v[pl.ds(h * (_HW // 2) + i * _L, _L)]
                for ch in range(_EMB_DIM):
                    vals = plsc.load_gather(cbt_v, [vidx + (ch * _NUM_EMB)])
                    out_v[ch, pl.ds(i * _L, _L)] = vals
                return carry

            lax.fori_loop(0, _HW // 2 // _L, chunk, 0)
            pltpu.sync_copy(
                out_v, out_hbm.at[wid, :, pl.ds(h * (_HW // 2), _HW // 2)])

        half(0)
        half(1)

    return sc_gather


def kernel(z_e, codebook):
    b, c, h, w = z_e.shape
    n_pos = b * h * w
    hw = h * w
    z3 = z_e.reshape(b, c, hw)

    idx3, loss_sum = _indices_and_loss(z3, codebook, b, c, hw)

    cbt = codebook.T.reshape(_EMB_DIM * _NUM_EMB)
    idx2 = idx3.reshape(b, hw)
    zq3 = _make_sc_gather(b)(cbt, idx2)

    loss = loss_sum[0, 0] * ((1.0 + _COMMIT) / (n_pos * _EMB_DIM))
    indices = idx3.reshape(n_pos)
    zq = zq3.reshape(b, c, h, w)
    return (zq, loss, indices)
